# baseline (speedup 1.0000x reference)
import functools

import jax
import jax.numpy as jnp
from jax import lax
from jax.experimental import pallas as pl
from jax.experimental.pallas import tpu as pltpu
from jax.experimental.pallas import tpu_sc as plsc

_H = 16
_M = 8 * 512 * 512
_NC, _NS = 2, 16
_NW = _NC * _NS
_PER_W = _M // _NW
_CH = 2048
_NCH = _PER_W // _CH

_mesh = plsc.VectorSubcoreMesh(core_axis_name="c", subcore_axis_name="s")


@functools.partial(
    pl.kernel,
    mesh=_mesh,
    compiler_params=pltpu.CompilerParams(use_tc_tiling_on_sc=False),
    out_type=jax.ShapeDtypeStruct((_M, _H), jnp.float32),
    scratch_types=[
        pltpu.VMEM((_CH,), jnp.int32),
        pltpu.VMEM((_CH, _H), jnp.float32),
        pltpu.SemaphoreType.DMA,
    ],
)
def _sc_lookup(dist_hbm, table_hbm, out_hbm, idx_v, rows_v, sem):
    wid = lax.axis_index("s") * _NC + lax.axis_index("c")
    base = wid * _PER_W

    def body(c, carry):
        off = base + c * _CH
        pltpu.sync_copy(dist_hbm.at[pl.ds(off, _CH)], idx_v)
        pltpu.async_copy(table_hbm.at[idx_v], rows_v, sem).wait()
        pltpu.sync_copy(rows_v, out_hbm.at[pl.ds(off, _CH)])
        return carry

    lax.fori_loop(0, _NCH, body, 0)


def kernel(dist, table):
    B, N, _ = dist.shape
    table_eff = table.at[0].set(0.0)
    out = _sc_lookup(dist.reshape(_M), table_eff)
    return out.reshape(B, N, N, _H)


# speedup vs baseline: 3.8870x; 3.8870x over previous
import functools

import jax
import jax.numpy as jnp
from jax import lax
from jax.experimental import pallas as pl
from jax.experimental.pallas import tpu as pltpu
from jax.experimental.pallas import tpu_sc as plsc

_H = 16
_K = 22
_M = 8 * 512 * 512
_NC, _NS = 2, 16
_NW = _NC * _NS
_PER_W = _M // _NW
_CH = 2048
_NCH = _PER_W // _CH
_UNROLL = 8

_mesh = plsc.VectorSubcoreMesh(core_axis_name="c", subcore_axis_name="s")


@functools.partial(
    pl.kernel,
    mesh=_mesh,
    compiler_params=pltpu.CompilerParams(use_tc_tiling_on_sc=False),
    out_type=jax.ShapeDtypeStruct((_M * _H,), jnp.float32),
    scratch_types=[
        pltpu.VMEM((_K * _H,), jnp.float32),
        pltpu.VMEM((_CH,), jnp.int32),
        pltpu.VMEM((_CH * _H,), jnp.float32),
        pltpu.SemaphoreType.DMA,
    ],
)
def _sc_lookup(dist_hbm, table_hbm, out_hbm, table_v, idx_v, out_v, sem):
    wid = lax.axis_index("s") * _NC + lax.axis_index("c")
    base = wid * _PER_W
    pltpu.sync_copy(table_hbm, table_v)

    def chunk_body(c, carry):
        off = base + c * _CH
        pltpu.sync_copy(dist_hbm.at[pl.ds(off, _CH)], idx_v)

        def row_body(r, carry2):
            r0 = r * _H
            d16 = idx_v[pl.ds(r0, _H)]
            for u in range(_H):
                out_v[pl.ds((r0 + u) * _H, _H)] = (
                    table_v[pl.ds(d16[u] * _H, _H)])
            return carry2

        lax.fori_loop(0, _CH // _H, row_body, 0)
        pltpu.sync_copy(out_v, out_hbm.at[pl.ds(off * _H, _CH * _H)])
        return carry

    lax.fori_loop(0, _NCH, chunk_body, 0)


def kernel(dist, table):
    B, N, _ = dist.shape
    table_eff = table.at[0].set(0.0)
    out = _sc_lookup(dist.reshape(_M), table_eff.reshape(_K * _H))
    return out.reshape(B, N, N, _H)


# speedup vs baseline: 4.9043x; 1.2617x over previous
import functools

import jax
import jax.numpy as jnp
from jax import lax
from jax.experimental import pallas as pl
from jax.experimental.pallas import tpu as pltpu
from jax.experimental.pallas import tpu_sc as plsc

_H = 16
_K = 22
_M = 8 * 512 * 512
_NC, _NS = 2, 16
_NW = _NC * _NS
_PER_W = _M // _NW
_CH = 2048
_NCH = _PER_W // _CH

_mesh = plsc.VectorSubcoreMesh(core_axis_name="c", subcore_axis_name="s")


@functools.partial(
    pl.kernel,
    mesh=_mesh,
    compiler_params=pltpu.CompilerParams(use_tc_tiling_on_sc=False),
    out_type=jax.ShapeDtypeStruct((_M, _H), jnp.float32),
    scratch_types=[
        pltpu.VMEM_SHARED((_K, _H), jnp.float32),
        pltpu.VMEM((_CH,), jnp.int32),
        pltpu.VMEM((_CH, _H), jnp.float32),
        pltpu.SemaphoreType.DMA,
    ],
)
def _sc_lookup(dist_hbm, table_hbm, out_hbm, table_v, idx_v, rows_v, sem):
    sid = lax.axis_index("s")
    wid = sid * _NC + lax.axis_index("c")
    base = wid * _PER_W

    @pl.when(sid == 0)
    def _load_table():
        pltpu.sync_copy(table_hbm, table_v)

    plsc.subcore_barrier()

    def chunk_body(c, carry):
        off = base + c * _CH
        pltpu.sync_copy(dist_hbm.at[pl.ds(off, _CH)], idx_v)
        pltpu.async_copy(table_v.at[idx_v], rows_v, sem).wait()
        pltpu.sync_copy(rows_v, out_hbm.at[pl.ds(off, _CH)])
        return carry

    lax.fori_loop(0, _NCH, chunk_body, 0)


def kernel(dist, table):
    B, N, _ = dist.shape
    table_eff = table.at[0].set(0.0)
    out = _sc_lookup(dist.reshape(_M), table_eff)
    return out.reshape(B, N, N, _H)
